# baseline (speedup 1.0000x reference)
import functools

import jax
import jax.numpy as jnp
from jax import lax
from jax.experimental import pallas as pl
from jax.experimental.pallas import tpu as pltpu
from jax.experimental.pallas import tpu_sc as plsc

NC = 2
NS = 16
LANES = 16
NW = NC * NS
CHUNK = 128
IB = 16


def _zero_rows(buf, nrows, d):
    def body(i, _):
        for k in range(d // LANES):
            buf[i, pl.ds(k * LANES, LANES)] = jnp.zeros((LANES,), jnp.float32)
        return 0
    lax.fori_loop(0, nrows, body, 0)


def _make_hop(n_pad, d, n_chunks):
    mesh = plsc.VectorSubcoreMesh(core_axis_name="c", subcore_axis_name="s")
    rows_per_sub = n_pad // NS
    grp = rows_per_sub // CHUNK

    nb = 2
    n_blk = n_chunks // IB

    @functools.partial(
        pl.kernel, mesh=mesh,
        out_type=jax.ShapeDtypeStruct((NC, n_pad, d), jnp.float32),
        compiler_params=pltpu.CompilerParams(use_tc_tiling_on_sc=False),
        scratch_types=[
            pltpu.VMEM((IB, CHUNK), jnp.int32),
            pltpu.VMEM((IB, CHUNK), jnp.int32),
            pltpu.VMEM((nb, CHUNK, d), jnp.float32),
            pltpu.VMEM_SHARED((n_pad, d), jnp.float32),
            pltpu.SemaphoreType.DMA,
            pltpu.SemaphoreType.DMA,
            pltpu.SemaphoreType.DMA,
        ])
    def hop(y_hbm, rowi_hbm, coli_hbm, out_hbm, idxr_v, idxc_v, rows_v,
            acc_sh, sem0, sem1, semg):
        sems = [sem0, sem1]
        c = lax.axis_index("c")
        s = lax.axis_index("s")
        wid = c * NS + s
        _zero_rows(rows_v.at[0], CHUNK, d)
        for t in range(grp):
            pltpu.sync_copy(rows_v.at[0],
                            acc_sh.at[pl.ds(s * rows_per_sub + t * CHUNK,
                                            CHUNK)])
        plsc.subcore_barrier()

        for kb in range(n_blk):
            pltpu.sync_copy(rowi_hbm.at[wid, pl.ds(kb * IB, IB)], idxr_v)
            pltpu.sync_copy(coli_hbm.at[wid, pl.ds(kb * IB, IB)], idxc_v)

            @pl.loop(0, IB, step=nb)
            def _chunks(g):
                for b in range(nb):
                    j = g + b
                    @pl.when(g > 0)
                    def _():
                        pltpu.make_async_copy(
                            rows_v.at[b], acc_sh.at[idxc_v.at[j]],
                            sems[b]).wait()
                    pltpu.async_copy(y_hbm.at[idxr_v.at[j]], rows_v.at[b],
                                     semg).wait()
                    pltpu.async_copy(rows_v.at[b], acc_sh.at[idxc_v.at[j]],
                                     sems[b], add=True)

            for b in range(nb):
                pltpu.make_async_copy(rows_v.at[b], acc_sh.at[idxc_v.at[b]],
                                      sems[b]).wait()

        plsc.subcore_barrier()
        for t in range(grp):
            off = s * rows_per_sub + t * CHUNK
            pltpu.sync_copy(acc_sh.at[pl.ds(off, CHUNK)],
                            out_hbm.at[c, pl.ds(off, CHUNK)])

    return hop


def _make_deg(n_pad, n_chunks):
    mesh = plsc.VectorSubcoreMesh(core_axis_name="c", subcore_axis_name="s")
    rows_per_sub = n_pad // NS
    grp = rows_per_sub // CHUNK

    @functools.partial(
        pl.kernel, mesh=mesh,
        out_type=jax.ShapeDtypeStruct((NC, n_pad, LANES), jnp.float32),
        compiler_params=pltpu.CompilerParams(use_tc_tiling_on_sc=False),
        scratch_types=[
            pltpu.VMEM((n_chunks, CHUNK), jnp.int32),
            pltpu.VMEM((CHUNK, LANES), jnp.float32),
            pltpu.VMEM_SHARED((n_pad, LANES), jnp.float32),
        ])
    def deg(coli_hbm, out_hbm, idxc_v, ones_v, acc_sh):
        c = lax.axis_index("c")
        s = lax.axis_index("s")
        wid = c * NS + s
        pltpu.sync_copy(coli_hbm.at[wid], idxc_v)
        _zero_rows(ones_v, CHUNK, LANES)
        for t in range(grp):
            pltpu.sync_copy(
                ones_v, acc_sh.at[pl.ds(s * rows_per_sub + t * CHUNK, CHUNK)])
        def fill(i, _):
            ones_v[i, pl.ds(0, LANES)] = jnp.ones((LANES,), jnp.float32)
            return 0
        lax.fori_loop(0, CHUNK, fill, 0)
        plsc.subcore_barrier()

        def body(j, _):
            pltpu.sync_copy(ones_v, acc_sh.at[idxc_v.at[j]], add=True)
            return 0
        lax.fori_loop(0, n_chunks, body, 0)
        plsc.subcore_barrier()
        for t in range(grp):
            off = s * rows_per_sub + t * CHUNK
            pltpu.sync_copy(acc_sh.at[pl.ds(off, CHUNK)],
                            out_hbm.at[c, pl.ds(off, CHUNK)])

    return deg


_BLK = 1024


def _prep_body(degp_ref, x_ref, deg_ref, y0_ref):
    dsum = degp_ref[0] + degp_ref[1]
    deg_ref[...] = dsum
    d1 = dsum[:, 0:1]
    dinv = jnp.where(d1 > 0, lax.rsqrt(d1), 0.0)
    y0_ref[...] = x_ref[...] * dinv


def _combine_body(p_ref, deg_ref, y_ref):
    d1 = deg_ref[...][:, 0:1]
    z = p_ref[0] + p_ref[1]
    y_ref[...] = jnp.where(d1 > 0, z / d1, 0.0)


def _mm1_body(x_ref, pa_ref, pb_ref, deg_ref, w_ref, b_ref, h_ref, yh_ref):
    d1 = deg_ref[...][:, 0:1]
    dinv = jnp.where(d1 > 0, lax.rsqrt(d1), 0.0)
    x1 = (pa_ref[0] + pa_ref[1]) * dinv
    x2 = (pb_ref[0] + pb_ref[1]) * dinv
    dd = x_ref.shape[1]
    acc = jnp.dot(x_ref[...], w_ref[0:dd], preferred_element_type=jnp.float32)
    acc += jnp.dot(x1, w_ref[dd:2 * dd], preferred_element_type=jnp.float32)
    acc += jnp.dot(x2, w_ref[2 * dd:3 * dd], preferred_element_type=jnp.float32)
    h = jnp.maximum(acc + b_ref[...], 0.0)
    h_ref[...] = h
    yh_ref[...] = h * dinv


def _mm2_body(h_ref, qa_ref, qb_ref, deg_ref, w_ref, b_ref, out_ref):
    d1 = deg_ref[...][:, 0:1]
    dinv = jnp.where(d1 > 0, lax.rsqrt(d1), 0.0)
    x1 = (qa_ref[0] + qa_ref[1]) * dinv
    x2 = (qb_ref[0] + qb_ref[1]) * dinv
    hh = h_ref.shape[1]
    acc = jnp.dot(h_ref[...], w_ref[0:hh], preferred_element_type=jnp.float32)
    acc += jnp.dot(x1, w_ref[hh:2 * hh], preferred_element_type=jnp.float32)
    acc += jnp.dot(x2, w_ref[2 * hh:3 * hh], preferred_element_type=jnp.float32)
    out_ref[...] = acc + b_ref[...]


def _row_spec(d):
    return pl.BlockSpec((_BLK, d), lambda i: (i, 0))


def _pair_spec(d):
    return pl.BlockSpec((NC, _BLK, d), lambda i: (0, i, 0))


def _full_spec(shape):
    return pl.BlockSpec(shape, lambda i: tuple(0 for _ in shape))


def kernel(x, edge_index, W1, b1, W2, b2):
    n, dd = x.shape
    hdim = W1.shape[1]
    e = edge_index.shape[1]

    n_pad = -(-n // (NS * CHUNK)) * (NS * CHUNK)
    e_pad = -(-e // (NW * CHUNK * IB)) * (NW * CHUNK * IB)
    n_chunks = e_pad // (NW * CHUNK)
    grid = n_pad // _BLK

    row = jnp.pad(edge_index[0], (0, e_pad - e))
    col = jnp.pad(edge_index[1], (0, e_pad - e),
                  constant_values=n)
    rowi = row.reshape(NW, n_chunks, CHUNK)
    coli = col.reshape(NW, n_chunks, CHUNK)
    x_pad = jnp.pad(x, ((0, n_pad - n), (0, 0)))

    hop_d = _make_hop(n_pad, dd, n_chunks)
    hop_h = _make_hop(n_pad, hdim, n_chunks)
    degk = _make_deg(n_pad, n_chunks)

    degp = degk(coli)

    deg, y0 = pl.pallas_call(
        _prep_body,
        grid=(grid,),
        in_specs=[_pair_spec(LANES), _row_spec(dd)],
        out_specs=[_row_spec(LANES), _row_spec(dd)],
        out_shape=[jax.ShapeDtypeStruct((n_pad, LANES), jnp.float32),
                   jax.ShapeDtypeStruct((n_pad, dd), jnp.float32)],
    )(degp, x_pad)

    def combine(p, d):
        return pl.pallas_call(
            _combine_body,
            grid=(grid,),
            in_specs=[_pair_spec(d), _row_spec(LANES)],
            out_specs=_row_spec(d),
            out_shape=jax.ShapeDtypeStruct((n_pad, d), jnp.float32),
        )(p, deg)

    P1 = hop_d(y0, rowi, coli)
    y1 = combine(P1, dd)
    P2 = hop_d(y1, rowi, coli)

    h, yh = pl.pallas_call(
        _mm1_body,
        grid=(grid,),
        in_specs=[_row_spec(dd), _pair_spec(dd), _pair_spec(dd),
                  _row_spec(LANES), _full_spec(W1.shape), _full_spec((1, hdim))],
        out_specs=[_row_spec(hdim), _row_spec(hdim)],
        out_shape=[jax.ShapeDtypeStruct((n_pad, hdim), jnp.float32),
                   jax.ShapeDtypeStruct((n_pad, hdim), jnp.float32)],
    )(x_pad, P1, P2, deg, W1, b1.reshape(1, hdim))

    Q1 = hop_h(yh, rowi, coli)
    y1h = combine(Q1, hdim)
    Q2 = hop_h(y1h, rowi, coli)

    out = pl.pallas_call(
        _mm2_body,
        grid=(grid,),
        in_specs=[_row_spec(hdim), _pair_spec(hdim), _pair_spec(hdim),
                  _row_spec(LANES), _full_spec(W2.shape), _full_spec((1, dd))],
        out_specs=_row_spec(dd),
        out_shape=jax.ShapeDtypeStruct((n_pad, dd), jnp.float32),
    )(h, Q1, Q2, deg, W2, b2.reshape(1, dd))

    return out[:n]


# speedup vs baseline: 2.8367x; 2.8367x over previous
import functools

import jax
import jax.numpy as jnp
from jax import lax
from jax.experimental import pallas as pl
from jax.experimental.pallas import tpu as pltpu
from jax.experimental.pallas import tpu_sc as plsc

NC = 2
NS = 16
LANES = 16
NW = NC * NS
CHUNK = 128
NB = 4


def _zero_rows(buf, nrows, d):
    def body(i, _):
        for k in range(d // LANES):
            buf[i, pl.ds(k * LANES, LANES)] = jnp.zeros((LANES,), jnp.float32)
        return 0
    lax.fori_loop(0, nrows, body, 0)


def _make_hop(n_pad, d, n_chunks):
    mesh = plsc.VectorSubcoreMesh(core_axis_name="c", subcore_axis_name="s")
    rows_per_sub = n_pad // NS
    grp = rows_per_sub // CHUNK

    @functools.partial(
        pl.kernel, mesh=mesh,
        out_type=jax.ShapeDtypeStruct((NC, n_pad, d), jnp.float32),
        compiler_params=pltpu.CompilerParams(use_tc_tiling_on_sc=False),
        scratch_types=[
            pltpu.VMEM((n_chunks, CHUNK), jnp.int32),
            pltpu.VMEM((n_chunks, CHUNK), jnp.int32),
            pltpu.VMEM((NB, CHUNK, d), jnp.float32),
            pltpu.VMEM_SHARED((n_pad, d), jnp.float32),
        ] + [pltpu.SemaphoreType.DMA] * NB)
    def hop(y_hbm, rowi_hbm, coli_hbm, out_hbm, idxr_v, idxc_v, rows_v,
            acc_sh, *sems):
        c = lax.axis_index("c")
        s = lax.axis_index("s")
        wid = c * NS + s
        pltpu.sync_copy(rowi_hbm.at[wid], idxr_v)
        pltpu.sync_copy(coli_hbm.at[wid], idxc_v)
        _zero_rows(rows_v.at[0], CHUNK, d)
        for t in range(grp):
            pltpu.sync_copy(rows_v.at[0],
                            acc_sh.at[pl.ds(s * rows_per_sub + t * CHUNK,
                                            CHUNK)])
        plsc.subcore_barrier()

        for b in range(NB):
            pltpu.async_copy(y_hbm.at[idxr_v.at[b]], rows_v.at[b], sems[b])

        @pl.loop(0, n_chunks, step=NB)
        def _chunks(g):
            for b in range(NB):
                j = g + b
                pltpu.make_async_copy(
                    y_hbm.at[idxr_v.at[j]], rows_v.at[b], sems[b]).wait()
                pltpu.sync_copy(rows_v.at[b], acc_sh.at[idxc_v.at[j]],
                                add=True)
                jn = j + NB

                @pl.when(jn < n_chunks)
                def _():
                    pltpu.async_copy(y_hbm.at[idxr_v.at[jn]], rows_v.at[b],
                                     sems[b])

        plsc.subcore_barrier()
        for t in range(grp):
            off = s * rows_per_sub + t * CHUNK
            pltpu.sync_copy(acc_sh.at[pl.ds(off, CHUNK)],
                            out_hbm.at[c, pl.ds(off, CHUNK)])

    return hop


def _make_deg(n_pad, n_chunks):
    mesh = plsc.VectorSubcoreMesh(core_axis_name="c", subcore_axis_name="s")
    rows_per_sub = n_pad // NS
    grp = rows_per_sub // CHUNK

    @functools.partial(
        pl.kernel, mesh=mesh,
        out_type=jax.ShapeDtypeStruct((NC, n_pad, LANES), jnp.float32),
        compiler_params=pltpu.CompilerParams(use_tc_tiling_on_sc=False),
        scratch_types=[
            pltpu.VMEM((n_chunks, CHUNK), jnp.int32),
            pltpu.VMEM((CHUNK, LANES), jnp.float32),
            pltpu.VMEM_SHARED((n_pad, LANES), jnp.float32),
        ])
    def deg(coli_hbm, out_hbm, idxc_v, ones_v, acc_sh):
        c = lax.axis_index("c")
        s = lax.axis_index("s")
        wid = c * NS + s
        pltpu.sync_copy(coli_hbm.at[wid], idxc_v)
        _zero_rows(ones_v, CHUNK, LANES)
        for t in range(grp):
            pltpu.sync_copy(
                ones_v, acc_sh.at[pl.ds(s * rows_per_sub + t * CHUNK, CHUNK)])
        def fill(i, _):
            ones_v[i, pl.ds(0, LANES)] = jnp.ones((LANES,), jnp.float32)
            return 0
        lax.fori_loop(0, CHUNK, fill, 0)
        plsc.subcore_barrier()

        def body(j, _):
            pltpu.sync_copy(ones_v, acc_sh.at[idxc_v.at[j]], add=True)
            return 0
        lax.fori_loop(0, n_chunks, body, 0)
        plsc.subcore_barrier()
        for t in range(grp):
            off = s * rows_per_sub + t * CHUNK
            pltpu.sync_copy(acc_sh.at[pl.ds(off, CHUNK)],
                            out_hbm.at[c, pl.ds(off, CHUNK)])

    return deg


_BLK = 1024


def _dinv_of(d1):
    return jnp.where(d1 > 0, lax.rsqrt(d1), 0.0)


def _proj_body(degp_ref, x_ref, w_ref, b_ref, deg_ref, xw_ref, y12_ref):
    dsum = degp_ref[0] + degp_ref[1]
    deg_ref[...] = dsum
    dinv = _dinv_of(dsum[:, 0:1])
    xb = x_ref[...]
    dd = xb.shape[1]
    hdim = w_ref.shape[1]
    xw_ref[...] = (
        jnp.dot(xb, w_ref[0:dd], preferred_element_type=jnp.float32)
        + b_ref[...])
    u1 = jnp.dot(xb, w_ref[dd:2 * dd], preferred_element_type=jnp.float32)
    u2 = jnp.dot(xb, w_ref[2 * dd:3 * dd], preferred_element_type=jnp.float32)
    y12_ref[:, 0:hdim] = u1 * dinv
    y12_ref[:, hdim:2 * hdim] = u2 * dinv


def _combine_hi_body(p_ref, deg_ref, y_ref):
    d1 = deg_ref[...][:, 0:1]
    hdim = y_ref.shape[1]
    z = p_ref[0, :, hdim:2 * hdim] + p_ref[1, :, hdim:2 * hdim]
    y_ref[...] = jnp.where(d1 > 0, z / d1, 0.0)


def _combine_body(p_ref, deg_ref, y_ref):
    d1 = deg_ref[...][:, 0:1]
    z = p_ref[0] + p_ref[1]
    y_ref[...] = jnp.where(d1 > 0, z / d1, 0.0)


def _relu_body(xw_ref, pab_ref, pc_ref, deg_ref, h_ref, yh_ref):
    dinv = _dinv_of(deg_ref[...][:, 0:1])
    hdim = h_ref.shape[1]
    z1 = pab_ref[0, :, 0:hdim] + pab_ref[1, :, 0:hdim]
    z2 = pc_ref[0] + pc_ref[1]
    h = jnp.maximum(xw_ref[...] + (z1 + z2) * dinv, 0.0)
    h_ref[...] = h
    yh_ref[...] = h * dinv


def _out_body(h_ref, qa_ref, qb_ref, deg_ref, w_ref, b_ref, out_ref):
    dinv = _dinv_of(deg_ref[...][:, 0:1])
    x1 = (qa_ref[0] + qa_ref[1]) * dinv
    x2 = (qb_ref[0] + qb_ref[1]) * dinv
    hh = h_ref.shape[1]
    acc = jnp.dot(h_ref[...], w_ref[0:hh], preferred_element_type=jnp.float32)
    acc += jnp.dot(x1, w_ref[hh:2 * hh], preferred_element_type=jnp.float32)
    acc += jnp.dot(x2, w_ref[2 * hh:3 * hh], preferred_element_type=jnp.float32)
    out_ref[...] = acc + b_ref[...]


def _row_spec(d):
    return pl.BlockSpec((_BLK, d), lambda i: (i, 0))


def _pair_spec(d):
    return pl.BlockSpec((NC, _BLK, d), lambda i: (0, i, 0))


def _full_spec(shape):
    return pl.BlockSpec(shape, lambda i: tuple(0 for _ in shape))


def kernel(x, edge_index, W1, b1, W2, b2):
    n, dd = x.shape
    hdim = W1.shape[1]
    e = edge_index.shape[1]

    n_pad = -(-n // (NS * CHUNK)) * (NS * CHUNK)
    e_pad = -(-e // (NW * CHUNK * NB)) * (NW * CHUNK * NB)
    n_chunks = e_pad // (NW * CHUNK)
    grid = n_pad // _BLK

    row = jnp.pad(edge_index[0], (0, e_pad - e))
    col = jnp.pad(edge_index[1], (0, e_pad - e),
                  constant_values=n)
    rowi = row.reshape(NW, n_chunks, CHUNK)
    coli = col.reshape(NW, n_chunks, CHUNK)
    x_pad = jnp.pad(x, ((0, n_pad - n), (0, 0)))

    hop16 = _make_hop(n_pad, hdim, n_chunks)
    hop32 = _make_hop(n_pad, 2 * hdim, n_chunks)
    degk = _make_deg(n_pad, n_chunks)

    degp = degk(coli)

    deg, xw, y12 = pl.pallas_call(
        _proj_body,
        grid=(grid,),
        in_specs=[_pair_spec(LANES), _row_spec(dd), _full_spec(W1.shape),
                  _full_spec((1, hdim))],
        out_specs=[_row_spec(LANES), _row_spec(hdim), _row_spec(2 * hdim)],
        out_shape=[jax.ShapeDtypeStruct((n_pad, LANES), jnp.float32),
                   jax.ShapeDtypeStruct((n_pad, hdim), jnp.float32),
                   jax.ShapeDtypeStruct((n_pad, 2 * hdim), jnp.float32)],
    )(degp, x_pad, W1, b1.reshape(1, hdim))

    Pab = hop32(y12, rowi, coli)

    yb = pl.pallas_call(
        _combine_hi_body,
        grid=(grid,),
        in_specs=[_pair_spec(2 * hdim), _row_spec(LANES)],
        out_specs=_row_spec(hdim),
        out_shape=jax.ShapeDtypeStruct((n_pad, hdim), jnp.float32),
    )(Pab, deg)

    Pc = hop16(yb, rowi, coli)

    h, yh = pl.pallas_call(
        _relu_body,
        grid=(grid,),
        in_specs=[_row_spec(hdim), _pair_spec(2 * hdim), _pair_spec(hdim),
                  _row_spec(LANES)],
        out_specs=[_row_spec(hdim), _row_spec(hdim)],
        out_shape=[jax.ShapeDtypeStruct((n_pad, hdim), jnp.float32),
                   jax.ShapeDtypeStruct((n_pad, hdim), jnp.float32)],
    )(xw, Pab, Pc, deg)

    Q1 = hop16(yh, rowi, coli)

    yv = pl.pallas_call(
        _combine_body,
        grid=(grid,),
        in_specs=[_pair_spec(hdim), _row_spec(LANES)],
        out_specs=_row_spec(hdim),
        out_shape=jax.ShapeDtypeStruct((n_pad, hdim), jnp.float32),
    )(Q1, deg)

    Q2 = hop16(yv, rowi, coli)

    out = pl.pallas_call(
        _out_body,
        grid=(grid,),
        in_specs=[_row_spec(hdim), _pair_spec(hdim), _pair_spec(hdim),
                  _row_spec(LANES), _full_spec(W2.shape), _full_spec((1, dd))],
        out_specs=_row_spec(dd),
        out_shape=jax.ShapeDtypeStruct((n_pad, dd), jnp.float32),
    )(h, Q1, Q2, deg, W2, b2.reshape(1, dd))

    return out[:n]


# speedup vs baseline: 3.1417x; 1.1075x over previous
import functools

import jax
import jax.numpy as jnp
from jax import lax
from jax.experimental import pallas as pl
from jax.experimental.pallas import tpu as pltpu
from jax.experimental.pallas import tpu_sc as plsc

NC = 2
NS = 16
LANES = 16
NW = NC * NS
CHUNK = 128
NB = 4


def _zero_rows(buf, nrows, d):
    def body(i, _):
        for k in range(d // LANES):
            buf[i, pl.ds(k * LANES, LANES)] = jnp.zeros((LANES,), jnp.float32)
        return 0
    lax.fori_loop(0, nrows, body, 0)


def _make_hop(n_pad, d, m0, m1):
    mesh = plsc.VectorSubcoreMesh(core_axis_name="c", subcore_axis_name="s")
    rows_per_sub = n_pad // NS
    grp = rows_per_sub // CHUNK
    mmax = max(m0, m1)

    @functools.partial(
        pl.kernel, mesh=mesh,
        out_type=jax.ShapeDtypeStruct((NC, n_pad, d), jnp.float32),
        compiler_params=pltpu.CompilerParams(use_tc_tiling_on_sc=False),
        scratch_types=[
            pltpu.VMEM((mmax, CHUNK), jnp.int32),
            pltpu.VMEM((mmax, CHUNK), jnp.int32),
            pltpu.VMEM((NB, CHUNK, d), jnp.float32),
            pltpu.VMEM_SHARED((n_pad, d), jnp.float32),
        ] + [pltpu.SemaphoreType.DMA] * NB)
    def hop(y_hbm, rowi_hbm, coli_hbm, out_hbm, idxr_v, idxc_v, rows_v,
            acc_sh, *sems):
        c = lax.axis_index("c")
        s = lax.axis_index("s")
        m = jnp.where(c == 0, m0, m1)
        base = jnp.where(c == 0, s * m0, NS * m0 + s * m1)
        pltpu.sync_copy(rowi_hbm.at[pl.ds(base, mmax)], idxr_v)
        pltpu.sync_copy(coli_hbm.at[pl.ds(base, mmax)], idxc_v)
        _zero_rows(rows_v.at[0], CHUNK, d)
        for t in range(grp):
            pltpu.sync_copy(rows_v.at[0],
                            acc_sh.at[pl.ds(s * rows_per_sub + t * CHUNK,
                                            CHUNK)])
        plsc.subcore_barrier()

        for b in range(NB):
            pltpu.async_copy(y_hbm.at[idxr_v.at[b]], rows_v.at[b], sems[b])

        @pl.loop(0, mmax, step=NB)
        def _chunks(g):
            @pl.when(g < m)
            def _():
                for b in range(NB):
                    j = g + b
                    pltpu.make_async_copy(
                        y_hbm.at[idxr_v.at[j]], rows_v.at[b], sems[b]).wait()
                    pltpu.sync_copy(rows_v.at[b], acc_sh.at[idxc_v.at[j]],
                                    add=True)
                    jn = j + NB

                    @pl.when(jn < m)
                    def _():
                        pltpu.async_copy(y_hbm.at[idxr_v.at[jn]],
                                         rows_v.at[b], sems[b])

        plsc.subcore_barrier()
        for t in range(grp):
            off = s * rows_per_sub + t * CHUNK
            pltpu.sync_copy(acc_sh.at[pl.ds(off, CHUNK)],
                            out_hbm.at[c, pl.ds(off, CHUNK)])

    return hop


def _make_deg(n_pad, n_chunks):
    mesh = plsc.VectorSubcoreMesh(core_axis_name="c", subcore_axis_name="s")
    rows_per_sub = n_pad // NS
    grp = rows_per_sub // CHUNK

    @functools.partial(
        pl.kernel, mesh=mesh,
        out_type=jax.ShapeDtypeStruct((NC, n_pad, LANES), jnp.float32),
        compiler_params=pltpu.CompilerParams(use_tc_tiling_on_sc=False),
        scratch_types=[
            pltpu.VMEM((n_chunks, CHUNK), jnp.int32),
            pltpu.VMEM((CHUNK, LANES), jnp.float32),
            pltpu.VMEM_SHARED((n_pad, LANES), jnp.float32),
        ])
    def deg(coli_hbm, out_hbm, idxc_v, ones_v, acc_sh):
        c = lax.axis_index("c")
        s = lax.axis_index("s")
        wid = c * NS + s
        pltpu.sync_copy(coli_hbm.at[pl.ds(wid * n_chunks, n_chunks)], idxc_v)
        _zero_rows(ones_v, CHUNK, LANES)
        for t in range(grp):
            pltpu.sync_copy(
                ones_v, acc_sh.at[pl.ds(s * rows_per_sub + t * CHUNK, CHUNK)])
        def fill(i, _):
            ones_v[i, pl.ds(0, LANES)] = jnp.ones((LANES,), jnp.float32)
            return 0
        lax.fori_loop(0, CHUNK, fill, 0)
        plsc.subcore_barrier()

        def body(j, _):
            pltpu.sync_copy(ones_v, acc_sh.at[idxc_v.at[j]], add=True)
            return 0
        lax.fori_loop(0, n_chunks, body, 0)
        plsc.subcore_barrier()
        for t in range(grp):
            off = s * rows_per_sub + t * CHUNK
            pltpu.sync_copy(acc_sh.at[pl.ds(off, CHUNK)],
                            out_hbm.at[c, pl.ds(off, CHUNK)])

    return deg


_BLK = 1024


def _dinv_of(d1):
    return jnp.where(d1 > 0, lax.rsqrt(d1), 0.0)


def _proj_body(degp_ref, x_ref, w_ref, b_ref, deg_ref, xw_ref, y12_ref):
    dsum = degp_ref[0] + degp_ref[1]
    deg_ref[...] = dsum
    dinv = _dinv_of(dsum[:, 0:1])
    xb = x_ref[...]
    dd = xb.shape[1]
    hdim = w_ref.shape[1]
    xw_ref[...] = (
        jnp.dot(xb, w_ref[0:dd], preferred_element_type=jnp.float32)
        + b_ref[...])
    u1 = jnp.dot(xb, w_ref[dd:2 * dd], preferred_element_type=jnp.float32)
    u2 = jnp.dot(xb, w_ref[2 * dd:3 * dd], preferred_element_type=jnp.float32)
    y12_ref[:, 0:hdim] = u1 * dinv
    y12_ref[:, hdim:2 * hdim] = u2 * dinv


def _combine_hi_body(p_ref, deg_ref, y_ref):
    d1 = deg_ref[...][:, 0:1]
    hdim = y_ref.shape[1]
    z = p_ref[0, :, hdim:2 * hdim] + p_ref[1, :, hdim:2 * hdim]
    y_ref[...] = jnp.where(d1 > 0, z / d1, 0.0)


def _combine_body(p_ref, deg_ref, y_ref):
    d1 = deg_ref[...][:, 0:1]
    z = p_ref[0] + p_ref[1]
    y_ref[...] = jnp.where(d1 > 0, z / d1, 0.0)


def _relu_body(xw_ref, pab_ref, pc_ref, deg_ref, h_ref, yh_ref):
    dinv = _dinv_of(deg_ref[...][:, 0:1])
    hdim = h_ref.shape[1]
    z1 = pab_ref[0, :, 0:hdim] + pab_ref[1, :, 0:hdim]
    z2 = pc_ref[0] + pc_ref[1]
    h = jnp.maximum(xw_ref[...] + (z1 + z2) * dinv, 0.0)
    h_ref[...] = h
    yh_ref[...] = h * dinv


def _out_body(h_ref, qa_ref, qb_ref, deg_ref, w_ref, b_ref, out_ref):
    dinv = _dinv_of(deg_ref[...][:, 0:1])
    x1 = (qa_ref[0] + qa_ref[1]) * dinv
    x2 = (qb_ref[0] + qb_ref[1]) * dinv
    hh = h_ref.shape[1]
    acc = jnp.dot(h_ref[...], w_ref[0:hh], preferred_element_type=jnp.float32)
    acc += jnp.dot(x1, w_ref[hh:2 * hh], preferred_element_type=jnp.float32)
    acc += jnp.dot(x2, w_ref[2 * hh:3 * hh], preferred_element_type=jnp.float32)
    out_ref[...] = acc + b_ref[...]


def _row_spec(d):
    return pl.BlockSpec((_BLK, d), lambda i: (i, 0))


def _pair_spec(d):
    return pl.BlockSpec((NC, _BLK, d), lambda i: (0, i, 0))


def _full_spec(shape):
    return pl.BlockSpec(shape, lambda i: tuple(0 for _ in shape))


def kernel(x, edge_index, W1, b1, W2, b2):
    n, dd = x.shape
    hdim = W1.shape[1]
    e = edge_index.shape[1]

    n_pad = -(-n // (NS * CHUNK)) * (NS * CHUNK)
    e_pad = -(-e // (NW * CHUNK * NB)) * (NW * CHUNK * NB)
    n_chunks = e_pad // (NW * CHUNK)
    mm = 2 * n_chunks
    m0_32, m1_32 = 120, 40
    m0_16, m1_16 = 108, 52
    assert m0_32 + m1_32 == mm and m0_16 + m1_16 == mm
    padc = max(m0_32, m0_16)
    grid = n_pad // _BLK

    row = jnp.pad(edge_index[0], (0, e_pad - e))
    col = jnp.pad(edge_index[1], (0, e_pad - e),
                  constant_values=n)
    rowi = jnp.pad(row.reshape(NW * n_chunks, CHUNK), ((0, padc), (0, 0)))
    coli = jnp.pad(col.reshape(NW * n_chunks, CHUNK), ((0, padc), (0, 0)),
                   constant_values=n)
    x_pad = jnp.pad(x, ((0, n_pad - n), (0, 0)))

    hop16 = _make_hop(n_pad, hdim, m0_16, m1_16)
    hop32 = _make_hop(n_pad, 2 * hdim, m0_32, m1_32)
    degk = _make_deg(n_pad, n_chunks)

    degp = degk(coli)

    deg, xw, y12 = pl.pallas_call(
        _proj_body,
        grid=(grid,),
        in_specs=[_pair_spec(LANES), _row_spec(dd), _full_spec(W1.shape),
                  _full_spec((1, hdim))],
        out_specs=[_row_spec(LANES), _row_spec(hdim), _row_spec(2 * hdim)],
        out_shape=[jax.ShapeDtypeStruct((n_pad, LANES), jnp.float32),
                   jax.ShapeDtypeStruct((n_pad, hdim), jnp.float32),
                   jax.ShapeDtypeStruct((n_pad, 2 * hdim), jnp.float32)],
    )(degp, x_pad, W1, b1.reshape(1, hdim))

    Pab = hop32(y12, rowi, coli)

    yb = pl.pallas_call(
        _combine_hi_body,
        grid=(grid,),
        in_specs=[_pair_spec(2 * hdim), _row_spec(LANES)],
        out_specs=_row_spec(hdim),
        out_shape=jax.ShapeDtypeStruct((n_pad, hdim), jnp.float32),
    )(Pab, deg)

    Pc = hop16(yb, rowi, coli)

    h, yh = pl.pallas_call(
        _relu_body,
        grid=(grid,),
        in_specs=[_row_spec(hdim), _pair_spec(2 * hdim), _pair_spec(hdim),
                  _row_spec(LANES)],
        out_specs=[_row_spec(hdim), _row_spec(hdim)],
        out_shape=[jax.ShapeDtypeStruct((n_pad, hdim), jnp.float32),
                   jax.ShapeDtypeStruct((n_pad, hdim), jnp.float32)],
    )(xw, Pab, Pc, deg)

    Q1 = hop16(yh, rowi, coli)

    yv = pl.pallas_call(
        _combine_body,
        grid=(grid,),
        in_specs=[_pair_spec(hdim), _row_spec(LANES)],
        out_specs=_row_spec(hdim),
        out_shape=jax.ShapeDtypeStruct((n_pad, hdim), jnp.float32),
    )(Q1, deg)

    Q2 = hop16(yv, rowi, coli)

    out = pl.pallas_call(
        _out_body,
        grid=(grid,),
        in_specs=[_row_spec(hdim), _pair_spec(hdim), _pair_spec(hdim),
                  _row_spec(LANES), _full_spec(W2.shape), _full_spec((1, dd))],
        out_specs=_row_spec(dd),
        out_shape=jax.ShapeDtypeStruct((n_pad, dd), jnp.float32),
    )(h, Q1, Q2, deg, W2, b2.reshape(1, dd))

    return out[:n]


# speedup vs baseline: 5.0531x; 1.6084x over previous
import functools

import jax
import jax.numpy as jnp
from jax import lax
from jax.experimental import pallas as pl
from jax.experimental.pallas import tpu as pltpu
from jax.experimental.pallas import tpu_sc as plsc

NC = 2
NS = 16
LANES = 16
NW = NC * NS
CHUNK = 128
NB = 4


def _zero_rows(buf, nrows, d):
    def body(i, _):
        for k in range(d // LANES):
            buf[i, pl.ds(k * LANES, LANES)] = jnp.zeros((LANES,), jnp.float32)
        return 0
    lax.fori_loop(0, nrows, body, 0)


def _hop_edges(ytab_sh, acc_sh, rowi_hbm, coli_hbm, out_hbm, idxr_v, idxc_v,
               rows_v, sems, c, s, m0, m1, mmax, n_pad, d):
    rows_per_sub = n_pad // NS
    grp = rows_per_sub // CHUNK
    m = jnp.where(c == 0, m0, m1)
    base = jnp.where(c == 0, s * m0, NS * m0 + s * m1)
    pltpu.sync_copy(rowi_hbm.at[pl.ds(base, mmax)], idxr_v)
    pltpu.sync_copy(coli_hbm.at[pl.ds(base, mmax)], idxc_v)
    _zero_rows(rows_v.at[0], CHUNK, d)
    for t in range(grp):
        pltpu.sync_copy(rows_v.at[0],
                        acc_sh.at[pl.ds(s * rows_per_sub + t * CHUNK, CHUNK)])
    plsc.subcore_barrier()

    for b in range(NB):
        pltpu.async_copy(ytab_sh.at[idxr_v.at[b]], rows_v.at[b], sems[b])

    @pl.loop(0, mmax, step=NB)
    def _chunks(g):
        @pl.when(g < m)
        def _():
            for b in range(NB):
                j = g + b
                pltpu.make_async_copy(
                    ytab_sh.at[idxr_v.at[j]], rows_v.at[b], sems[b]).wait()
                pltpu.sync_copy(rows_v.at[b], acc_sh.at[idxc_v.at[j]],
                                add=True)
                jn = j + NB

                @pl.when(jn < m)
                def _():
                    pltpu.async_copy(ytab_sh.at[idxr_v.at[jn]],
                                     rows_v.at[b], sems[b])

    plsc.subcore_barrier()
    for t in range(grp):
        off = s * rows_per_sub + t * CHUNK
        pltpu.sync_copy(acc_sh.at[pl.ds(off, CHUNK)],
                        out_hbm.at[c, pl.ds(off, CHUNK)])


def _hop_scratch(n_pad, d, mmax):
    return [
        pltpu.VMEM((mmax, CHUNK), jnp.int32),
        pltpu.VMEM((mmax, CHUNK), jnp.int32),
        pltpu.VMEM((NB, CHUNK, d), jnp.float32),
        pltpu.VMEM_SHARED((n_pad, d), jnp.float32),
        pltpu.VMEM_SHARED((n_pad, d), jnp.float32),
    ] + [pltpu.SemaphoreType.DMA] * NB


_MESH = dict(core_axis_name="c", subcore_axis_name="s")


def _make_hop_first(n_pad, d, m0, m1):
    mmax = max(m0, m1)
    rows_per_sub = n_pad // NS

    @functools.partial(
        pl.kernel, mesh=plsc.VectorSubcoreMesh(**_MESH),
        out_type=jax.ShapeDtypeStruct((NC, n_pad, d), jnp.float32),
        compiler_params=pltpu.CompilerParams(use_tc_tiling_on_sc=False),
        scratch_types=_hop_scratch(n_pad, d, mmax))
    def hop(y_hbm, rowi_hbm, coli_hbm, out_hbm, idxr_v, idxc_v, rows_v,
            ytab_sh, acc_sh, *sems):
        c = lax.axis_index("c")
        s = lax.axis_index("s")
        off = s * rows_per_sub
        pltpu.sync_copy(y_hbm.at[pl.ds(off, rows_per_sub)],
                        ytab_sh.at[pl.ds(off, rows_per_sub)])
        _hop_edges(ytab_sh, acc_sh, rowi_hbm, coli_hbm, out_hbm, idxr_v,
                   idxc_v, rows_v, sems, c, s, m0, m1, mmax, n_pad, d)

    return hop


def _make_hop_mid(n_pad, d, m0, m1, lo_col):
    mmax = max(m0, m1)
    rows_per_sub = n_pad // NS
    grp = rows_per_sub // CHUNK

    @functools.partial(
        pl.kernel, mesh=plsc.VectorSubcoreMesh(**_MESH),
        out_type=jax.ShapeDtypeStruct((NC, n_pad, d), jnp.float32),
        compiler_params=pltpu.CompilerParams(use_tc_tiling_on_sc=False),
        scratch_types=[
            pltpu.VMEM((CHUNK, d), jnp.float32),
            pltpu.VMEM((CHUNK, d), jnp.float32),
            pltpu.VMEM((CHUNK, LANES), jnp.float32),
            pltpu.VMEM((CHUNK, d), jnp.float32),
        ] + _hop_scratch(n_pad, d, mmax))
    def hop(p_hbm, invd_hbm, rowi_hbm, coli_hbm, out_hbm, sa, sb, sd, yb,
            idxr_v, idxc_v, rows_v, ytab_sh, acc_sh, *sems):
        c = lax.axis_index("c")
        s = lax.axis_index("s")
        for t in range(grp):
            off = s * rows_per_sub + t * CHUNK
            pltpu.sync_copy(
                p_hbm.at[0, pl.ds(off, CHUNK), pl.ds(lo_col, d)], sa)
            pltpu.sync_copy(
                p_hbm.at[1, pl.ds(off, CHUNK), pl.ds(lo_col, d)], sb)
            pltpu.sync_copy(invd_hbm.at[pl.ds(off, CHUNK)], sd)

            def rowbody(i, _):
                for k in range(d // LANES):
                    sl = pl.ds(k * LANES, LANES)
                    yb[i, sl] = (sa[i, sl] + sb[i, sl]) * sd[i, :]
                return 0
            lax.fori_loop(0, CHUNK, rowbody, 0)
            pltpu.sync_copy(yb, ytab_sh.at[pl.ds(off, CHUNK)])
        _hop_edges(ytab_sh, acc_sh, rowi_hbm, coli_hbm, out_hbm, idxr_v,
                   idxc_v, rows_v, sems, c, s, m0, m1, mmax, n_pad, d)

    return hop


def _make_hop_relu(n_pad, d, m0, m1):
    mmax = max(m0, m1)
    rows_per_sub = n_pad // NS
    grp = rows_per_sub // CHUNK

    @functools.partial(
        pl.kernel, mesh=plsc.VectorSubcoreMesh(**_MESH),
        out_type=(jax.ShapeDtypeStruct((NC, n_pad, d), jnp.float32),
                  jax.ShapeDtypeStruct((n_pad, d), jnp.float32)),
        compiler_params=pltpu.CompilerParams(use_tc_tiling_on_sc=False),
        scratch_types=[
            pltpu.VMEM((CHUNK, d), jnp.float32),
            pltpu.VMEM((CHUNK, d), jnp.float32),
            pltpu.VMEM((CHUNK, d), jnp.float32),
            pltpu.VMEM((CHUNK, d), jnp.float32),
            pltpu.VMEM((CHUNK, d), jnp.float32),
            pltpu.VMEM((CHUNK, LANES), jnp.float32),
            pltpu.VMEM((CHUNK, d), jnp.float32),
            pltpu.VMEM((CHUNK, d), jnp.float32),
        ] + _hop_scratch(n_pad, d, mmax))
    def hop(pab_hbm, pc_hbm, xw_hbm, dinv_hbm, rowi_hbm, coli_hbm,
            out_hbm, h_hbm, sa, sb, sc0, sc1, sx, sd, yb, hb,
            idxr_v, idxc_v, rows_v, ytab_sh, acc_sh, *sems):
        c = lax.axis_index("c")
        s = lax.axis_index("s")
        for t in range(grp):
            off = s * rows_per_sub + t * CHUNK
            pltpu.sync_copy(pab_hbm.at[0, pl.ds(off, CHUNK), pl.ds(0, d)], sa)
            pltpu.sync_copy(pab_hbm.at[1, pl.ds(off, CHUNK), pl.ds(0, d)], sb)
            pltpu.sync_copy(pc_hbm.at[0, pl.ds(off, CHUNK)], sc0)
            pltpu.sync_copy(pc_hbm.at[1, pl.ds(off, CHUNK)], sc1)
            pltpu.sync_copy(xw_hbm.at[pl.ds(off, CHUNK)], sx)
            pltpu.sync_copy(dinv_hbm.at[pl.ds(off, CHUNK)], sd)

            def rowbody(i, _):
                for k in range(d // LANES):
                    sl = pl.ds(k * LANES, LANES)
                    z = sa[i, sl] + sb[i, sl] + sc0[i, sl] + sc1[i, sl]
                    h = jnp.maximum(sx[i, sl] + z * sd[i, :], 0.0)
                    hb[i, sl] = h
                    yb[i, sl] = h * sd[i, :]
                return 0
            lax.fori_loop(0, CHUNK, rowbody, 0)
            pltpu.sync_copy(hb, h_hbm.at[pl.ds(off, CHUNK)])
            pltpu.sync_copy(yb, ytab_sh.at[pl.ds(off, CHUNK)])
        _hop_edges(ytab_sh, acc_sh, rowi_hbm, coli_hbm, out_hbm, idxr_v,
                   idxc_v, rows_v, sems, c, s, m0, m1, mmax, n_pad, d)

    return hop


def _make_deg(n_pad, n_chunks):
    rows_per_sub = n_pad // NS
    grp = rows_per_sub // CHUNK

    @functools.partial(
        pl.kernel, mesh=plsc.VectorSubcoreMesh(**_MESH),
        out_type=jax.ShapeDtypeStruct((NC, n_pad, LANES), jnp.float32),
        compiler_params=pltpu.CompilerParams(use_tc_tiling_on_sc=False),
        scratch_types=[
            pltpu.VMEM((n_chunks, CHUNK), jnp.int32),
            pltpu.VMEM((CHUNK, LANES), jnp.float32),
            pltpu.VMEM_SHARED((n_pad, LANES), jnp.float32),
        ])
    def deg(coli_hbm, out_hbm, idxc_v, ones_v, acc_sh):
        c = lax.axis_index("c")
        s = lax.axis_index("s")
        wid = c * NS + s
        pltpu.sync_copy(coli_hbm.at[pl.ds(wid * n_chunks, n_chunks)], idxc_v)
        _zero_rows(ones_v, CHUNK, LANES)
        for t in range(grp):
            pltpu.sync_copy(
                ones_v, acc_sh.at[pl.ds(s * rows_per_sub + t * CHUNK, CHUNK)])
        def fill(i, _):
            ones_v[i, pl.ds(0, LANES)] = jnp.ones((LANES,), jnp.float32)
            return 0
        lax.fori_loop(0, CHUNK, fill, 0)
        plsc.subcore_barrier()

        def body(j, _):
            pltpu.sync_copy(ones_v, acc_sh.at[idxc_v.at[j]], add=True)
            return 0
        lax.fori_loop(0, n_chunks, body, 0)
        plsc.subcore_barrier()
        for t in range(grp):
            off = s * rows_per_sub + t * CHUNK
            pltpu.sync_copy(acc_sh.at[pl.ds(off, CHUNK)],
                            out_hbm.at[c, pl.ds(off, CHUNK)])

    return deg


_BLK = 1024


def _proj_body(degp_ref, x_ref, w_ref, b_ref, xw_ref, y12_ref, dinv_ref,
               invd_ref):
    dsum = degp_ref[0] + degp_ref[1]
    pos = dsum > 0
    dinv = jnp.where(pos, lax.rsqrt(dsum), 0.0)
    dinv_ref[...] = dinv
    invd_ref[...] = jnp.where(pos, 1.0 / dsum, 0.0)
    d1 = dinv[:, 0:1]
    xb = x_ref[...]
    dd = xb.shape[1]
    hdim = xw_ref.shape[1]
    xw_ref[...] = (
        jnp.dot(xb, w_ref[0:dd], preferred_element_type=jnp.float32)
        + b_ref[...])
    u1 = jnp.dot(xb, w_ref[dd:2 * dd], preferred_element_type=jnp.float32)
    u2 = jnp.dot(xb, w_ref[2 * dd:3 * dd], preferred_element_type=jnp.float32)
    y12_ref[:, 0:hdim] = u1 * d1
    y12_ref[:, hdim:2 * hdim] = u2 * d1


def _out_body(h_ref, qa_ref, qb_ref, dinv_ref, w_ref, b_ref, out_ref):
    d1 = dinv_ref[...][:, 0:1]
    x1 = (qa_ref[0] + qa_ref[1]) * d1
    x2 = (qb_ref[0] + qb_ref[1]) * d1
    hh = h_ref.shape[1]
    acc = jnp.dot(h_ref[...], w_ref[0:hh], preferred_element_type=jnp.float32)
    acc += jnp.dot(x1, w_ref[hh:2 * hh], preferred_element_type=jnp.float32)
    acc += jnp.dot(x2, w_ref[2 * hh:3 * hh], preferred_element_type=jnp.float32)
    out_ref[...] = acc + b_ref[...]


def _row_spec(d):
    return pl.BlockSpec((_BLK, d), lambda i: (i, 0))


def _pair_spec(d):
    return pl.BlockSpec((NC, _BLK, d), lambda i: (0, i, 0))


def _full_spec(shape):
    return pl.BlockSpec(shape, lambda i: tuple(0 for _ in shape))


def kernel(x, edge_index, W1, b1, W2, b2):
    n, dd = x.shape
    hdim = W1.shape[1]
    e = edge_index.shape[1]

    n_pad = -(-n // (NS * CHUNK)) * (NS * CHUNK)
    e_pad = -(-e // (NW * CHUNK * NB)) * (NW * CHUNK * NB)
    n_chunks = e_pad // (NW * CHUNK)
    mm = 2 * n_chunks
    m0_32, m1_32 = 80, 80
    m0_16, m1_16 = 80, 80
    assert m0_32 + m1_32 == mm and m0_16 + m1_16 == mm
    padc = max(m0_32, m0_16, m1_32, m1_16)
    grid = n_pad // _BLK

    row = jnp.pad(edge_index[0], (0, e_pad - e))
    col = jnp.pad(edge_index[1], (0, e_pad - e),
                  constant_values=n)
    rowi = jnp.pad(row.reshape(NW * n_chunks, CHUNK), ((0, padc), (0, 0)))
    coli = jnp.pad(col.reshape(NW * n_chunks, CHUNK), ((0, padc), (0, 0)),
                   constant_values=n)
    x_pad = jnp.pad(x, ((0, n_pad - n), (0, 0)))

    hopA = _make_hop_first(n_pad, 2 * hdim, m0_32, m1_32)
    hopC = _make_hop_mid(n_pad, hdim, m0_16, m1_16, hdim)
    hopD = _make_hop_relu(n_pad, hdim, m0_16, m1_16)
    hopE = _make_hop_mid(n_pad, hdim, m0_16, m1_16, 0)
    degk = _make_deg(n_pad, n_chunks)

    degp = degk(coli)

    xw, y12, dinv, invd = pl.pallas_call(
        _proj_body,
        grid=(grid,),
        in_specs=[_pair_spec(LANES), _row_spec(dd), _full_spec(W1.shape),
                  _full_spec((1, hdim))],
        out_specs=[_row_spec(hdim), _row_spec(2 * hdim), _row_spec(LANES),
                   _row_spec(LANES)],
        out_shape=[jax.ShapeDtypeStruct((n_pad, hdim), jnp.float32),
                   jax.ShapeDtypeStruct((n_pad, 2 * hdim), jnp.float32),
                   jax.ShapeDtypeStruct((n_pad, LANES), jnp.float32),
                   jax.ShapeDtypeStruct((n_pad, LANES), jnp.float32)],
    )(degp, x_pad, W1, b1.reshape(1, hdim))

    Pab = hopA(y12, rowi, coli)
    Pc = hopC(Pab, invd, rowi, coli)
    Q1, h = hopD(Pab, Pc, xw, dinv, rowi, coli)
    Q2 = hopE(Q1, invd, rowi, coli)

    out = pl.pallas_call(
        _out_body,
        grid=(grid,),
        in_specs=[_row_spec(hdim), _pair_spec(hdim), _pair_spec(hdim),
                  _row_spec(LANES), _full_spec(W2.shape), _full_spec((1, dd))],
        out_specs=_row_spec(dd),
        out_shape=jax.ShapeDtypeStruct((n_pad, dd), jnp.float32),
    )(h, Q1, Q2, dinv, W2, b2.reshape(1, dd))

    return out[:n]
